# baseline (speedup 1.0000x reference)
import functools

import jax
import jax.numpy as jnp
from jax import lax
from jax.experimental import pallas as pl
from jax.experimental.pallas import tpu as pltpu
from jax.experimental.pallas import tpu_sc as plsc

BS = 4096
NUMCANDS = 20
EDIM = 64
NUMWORDS = 10000
NUMENS = 1000000

_NC = 2
_NS = 16
_NW = _NC * _NS
_ROWS = BS * NUMCANDS
_ROWS_PER_W = _ROWS // _NW
_CHUNK = 128
_NCHUNK = _ROWS_PER_W // _CHUNK


def _gather_body(idx_hbm, table_hbm, out_hbm, idx_v, rows_v, gsem, ssem):
    c = lax.axis_index("c")
    s = lax.axis_index("s")
    wid = s * _NC + c
    pltpu.sync_copy(idx_hbm.at[wid], idx_v)
    base = wid * _ROWS_PER_W
    pltpu.async_copy(table_hbm.at[idx_v.at[0]], rows_v.at[0], gsem)
    for j in range(_NCHUNK):
        if j + 1 < _NCHUNK:
            pltpu.async_copy(
                table_hbm.at[idx_v.at[j + 1]], rows_v.at[(j + 1) % 2], gsem)
        pltpu.make_async_copy(
            table_hbm.at[idx_v.at[j]], rows_v.at[j % 2], gsem).wait()
        pltpu.async_copy(
            rows_v.at[j % 2], out_hbm.at[pl.ds(base + j * _CHUNK, _CHUNK)], ssem)
        if j >= 1:
            pltpu.make_async_copy(
                rows_v.at[(j - 1) % 2],
                out_hbm.at[pl.ds(base + (j - 1) * _CHUNK, _CHUNK)], ssem).wait()
    pltpu.make_async_copy(
        rows_v.at[(_NCHUNK - 1) % 2],
        out_hbm.at[pl.ds(base + (_NCHUNK - 1) * _CHUNK, _CHUNK)], ssem).wait()


_gather = functools.partial(
    pl.kernel,
    out_type=jax.ShapeDtypeStruct((_ROWS, EDIM), jnp.float32),
    mesh=plsc.VectorSubcoreMesh(core_axis_name="c", subcore_axis_name="s"),
    scratch_types=[
        pltpu.VMEM((_NCHUNK, _CHUNK), jnp.int32),
        pltpu.VMEM((2, _CHUNK, EDIM), jnp.float32),
        pltpu.SemaphoreType.DMA,
        pltpu.SemaphoreType.DMA,
    ],
    compiler_params=pltpu.CompilerParams(use_tc_tiling_on_sc=False),
)(_gather_body)


_BB = 256


def _ctx_body(docT_ref, docmat_ref, ctx_ref):
    ctx_ref[...] = lax.dot_general(
        docT_ref[...], docmat_ref[...],
        dimension_numbers=(((0,), (0,)), ((), ())),
        preferred_element_type=jnp.float32)


def _score_body(ctx_ref, emb_ref, scoresT_ref, probsT_ref):
    ctx = ctx_ref[...]
    emb = emb_ref[...]
    sT = jnp.sum(ctx[None, :, :] * emb, axis=2)
    scoresT_ref[...] = sT
    m = jnp.max(sT, axis=0, keepdims=True)
    e = jnp.exp(sT - m)
    probsT_ref[...] = e / jnp.sum(e, axis=0, keepdims=True)


def kernel(cands, doc, entity_table, docmat):
    docT = doc.T
    idx = cands.T.astype(jnp.int32).reshape(_NW, _NCHUNK, _CHUNK)

    emb = _gather(idx, entity_table)

    ctx = pl.pallas_call(
        _ctx_body,
        grid=(BS // _BB,),
        in_specs=[
            pl.BlockSpec((NUMWORDS, _BB), lambda i: (0, i)),
            pl.BlockSpec((NUMWORDS, EDIM), lambda i: (0, 0)),
        ],
        out_specs=pl.BlockSpec((_BB, EDIM), lambda i: (i, 0)),
        out_shape=jax.ShapeDtypeStruct((BS, EDIM), jnp.float32),
    )(docT, docmat)

    emb3 = emb.reshape(NUMCANDS, BS, EDIM)
    scoresT, probsT = pl.pallas_call(
        _score_body,
        grid=(BS // _BB,),
        in_specs=[
            pl.BlockSpec((_BB, EDIM), lambda i: (i, 0)),
            pl.BlockSpec((NUMCANDS, _BB, EDIM), lambda i: (0, i, 0)),
        ],
        out_specs=[
            pl.BlockSpec((NUMCANDS, _BB), lambda i: (0, i)),
            pl.BlockSpec((NUMCANDS, _BB), lambda i: (0, i)),
        ],
        out_shape=[
            jax.ShapeDtypeStruct((NUMCANDS, BS), jnp.float32),
            jax.ShapeDtypeStruct((NUMCANDS, BS), jnp.float32),
        ],
    )(ctx, emb3)
    return scoresT.T, probsT.T


# speedup vs baseline: 1.5866x; 1.5866x over previous
import functools

import jax
import jax.numpy as jnp
from jax import lax
from jax.experimental import pallas as pl
from jax.experimental.pallas import tpu as pltpu
from jax.experimental.pallas import tpu_sc as plsc

BS = 4096
NUMCANDS = 20
EDIM = 64
NUMWORDS = 10000
NUMENS = 1000000

_NC = 2
_NS = 16
_NW = _NC * _NS
_ROWS = BS * NUMCANDS
_ROWS_PER_W = _ROWS // _NW
_CHUNK = 64
_NCHUNK = _ROWS_PER_W // _CHUNK


def _gather_body(idx_hbm, table_hbm, out_hbm, idx_v, rows_v, gsem, ssem):
    c = lax.axis_index("c")
    s = lax.axis_index("s")
    wid = s * _NC + c
    base = wid * _ROWS_PER_W
    pltpu.sync_copy(idx_hbm.at[pl.ds(base, _ROWS_PER_W)], idx_v)

    def fire(chunk, buf):
        off = chunk * _CHUNK
        for g in range(_CHUNK // 16):
            vec = idx_v[pl.ds(off + g * 16, 16)]
            for k in range(16):
                pltpu.async_copy(
                    table_hbm.at[pl.ds(vec[k], 1)],
                    rows_v.at[pl.ds(buf, 1)].at[0, pl.ds(g * 16 + k, 1)], gsem)

    def drain(chunk, buf):
        dummy = rows_v.at[pl.ds(buf, 1)].at[0, pl.ds(0, 1)]
        for k in range(_CHUNK):
            pltpu.make_async_copy(table_hbm.at[pl.ds(0, 1)], dummy, gsem).wait()

    def store_start(chunk, buf):
        pltpu.async_copy(
            rows_v.at[pl.ds(buf, 1)].at[0],
            out_hbm.at[pl.ds(base + chunk * _CHUNK, _CHUNK)], ssem)

    def store_wait(chunk, buf):
        pltpu.make_async_copy(
            rows_v.at[pl.ds(buf, 1)].at[0],
            out_hbm.at[pl.ds(base + chunk * _CHUNK, _CHUNK)], ssem).wait()

    fire(0, 0)

    def body(i, _):
        buf = lax.rem(i, 2)
        nbuf = 1 - buf

        @pl.when(i + 1 < _NCHUNK)
        def _():
            fire(i + 1, nbuf)

        drain(i, buf)

        @pl.when(i >= 2)
        def _():
            store_wait(i - 2, buf)

        store_start(i, buf)
        return ()

    lax.fori_loop(0, _NCHUNK, body, ())
    store_wait(_NCHUNK - 2, (_NCHUNK - 2) % 2)
    store_wait(_NCHUNK - 1, (_NCHUNK - 1) % 2)


_gather = functools.partial(
    pl.kernel,
    out_type=jax.ShapeDtypeStruct((_ROWS, EDIM), jnp.float32),
    mesh=plsc.VectorSubcoreMesh(core_axis_name="c", subcore_axis_name="s"),
    scratch_types=[
        pltpu.VMEM((_ROWS_PER_W,), jnp.int32),
        pltpu.VMEM((2, _CHUNK, EDIM), jnp.float32),
        pltpu.SemaphoreType.DMA,
        pltpu.SemaphoreType.DMA,
    ],
)(_gather_body)


_BB = 256


def _ctx_body(docT_ref, docmat_ref, ctx_ref):
    ctx_ref[...] = lax.dot_general(
        docT_ref[...], docmat_ref[...],
        dimension_numbers=(((0,), (0,)), ((), ())),
        preferred_element_type=jnp.float32)


def _score_body(ctx_ref, emb_ref, scoresT_ref, probsT_ref):
    ctx = ctx_ref[...]
    emb = emb_ref[...]
    sT = jnp.sum(ctx[None, :, :] * emb, axis=2)
    scoresT_ref[...] = sT
    m = jnp.max(sT, axis=0, keepdims=True)
    e = jnp.exp(sT - m)
    probsT_ref[...] = e / jnp.sum(e, axis=0, keepdims=True)


def kernel(cands, doc, entity_table, docmat):
    docT = doc.T
    idx = cands.T.astype(jnp.int32).reshape(_ROWS)

    emb = _gather(idx, entity_table)

    ctx = pl.pallas_call(
        _ctx_body,
        grid=(BS // _BB,),
        in_specs=[
            pl.BlockSpec((NUMWORDS, _BB), lambda i: (0, i)),
            pl.BlockSpec((NUMWORDS, EDIM), lambda i: (0, 0)),
        ],
        out_specs=pl.BlockSpec((_BB, EDIM), lambda i: (i, 0)),
        out_shape=jax.ShapeDtypeStruct((BS, EDIM), jnp.float32),
    )(docT, docmat)

    emb3 = emb.reshape(NUMCANDS, BS, EDIM)
    scoresT, probsT = pl.pallas_call(
        _score_body,
        grid=(BS // _BB,),
        in_specs=[
            pl.BlockSpec((_BB, EDIM), lambda i: (i, 0)),
            pl.BlockSpec((NUMCANDS, _BB, EDIM), lambda i: (0, i, 0)),
        ],
        out_specs=[
            pl.BlockSpec((NUMCANDS, _BB), lambda i: (0, i)),
            pl.BlockSpec((NUMCANDS, _BB), lambda i: (0, i)),
        ],
        out_shape=[
            jax.ShapeDtypeStruct((NUMCANDS, BS), jnp.float32),
            jax.ShapeDtypeStruct((NUMCANDS, BS), jnp.float32),
        ],
    )(ctx, emb3)
    return scoresT.T, probsT.T
